# Initial kernel scaffold; baseline (speedup 1.0000x reference)
import jax
import jax.numpy as jnp
from jax.experimental import pallas as pl


def kernel(x, W, b):
    raise NotImplementedError("write your pallas kernel here")


# speedup vs baseline: 6.0869x; 6.0869x over previous
import jax
import jax.numpy as jnp
from jax.experimental import pallas as pl

N_EXPERT = 64
TOKEN_BLOCK = 2048
_NEG_INF = float("-inf")


def _router_block(x_ref, w_ref, b_ref, out_ref, ids_ref):
    x = x_ref[...]
    logits = jax.lax.dot_general(
        x, w_ref[...], (((1,), (1,)), ((), ())),
        preferred_element_type=jnp.float32,
    ) + b_ref[...]
    t = logits.shape[0]
    idx = jax.lax.broadcasted_iota(jnp.int32, (t, N_EXPERT), 1)
    big = jnp.int32(N_EXPERT)

    m1 = jnp.max(logits, axis=1, keepdims=True)
    id1 = jnp.min(jnp.where(logits == m1, idx, big), axis=1, keepdims=True)
    masked = jnp.where(idx == id1, _NEG_INF, logits)
    m2 = jnp.max(masked, axis=1, keepdims=True)
    id2 = jnp.min(jnp.where(masked == m2, idx, big), axis=1, keepdims=True)

    s = jnp.exp(m2 - m1)
    denom = 1.0 + s
    p1 = 1.0 / denom
    p2 = s / denom

    out_ref[...] = jnp.where(idx == id1, p1, 0.0) + jnp.where(idx == id2, p2, 0.0)
    ids_ref[...] = jnp.concatenate([id1, id2], axis=1)


def kernel(x, W, b):
    B, S, D = x.shape
    n_tokens = B * S
    xs = x.reshape(n_tokens, D)
    b2 = b.reshape(1, N_EXPERT)
    grid = (n_tokens // TOKEN_BLOCK,)
    out, ids = pl.pallas_call(
        _router_block,
        grid=grid,
        in_specs=[
            pl.BlockSpec((TOKEN_BLOCK, D), lambda i: (i, 0)),
            pl.BlockSpec((N_EXPERT, D), lambda i: (0, 0)),
            pl.BlockSpec((1, N_EXPERT), lambda i: (0, 0)),
        ],
        out_specs=[
            pl.BlockSpec((TOKEN_BLOCK, N_EXPERT), lambda i: (i, 0)),
            pl.BlockSpec((TOKEN_BLOCK, 2), lambda i: (i, 0)),
        ],
        out_shape=[
            jax.ShapeDtypeStruct((n_tokens, N_EXPERT), jnp.float32),
            jax.ShapeDtypeStruct((n_tokens, 2), jnp.int32),
        ],
    )(xs, W, b2)
    return out.reshape(B, S, N_EXPERT), ids.reshape(B, S, 2)


# speedup vs baseline: 6.4872x; 1.0658x over previous
import jax
import jax.numpy as jnp
from jax.experimental import pallas as pl

N_EXPERT = 64
TOKEN_BLOCK = 4096
_NEG_INF = float("-inf")


def _router_block(x_ref, w_ref, b_ref, out_ref, ids_ref):
    x = x_ref[...]
    logits = jax.lax.dot_general(
        x, w_ref[...], (((1,), (1,)), ((), ())),
        preferred_element_type=jnp.float32,
    ) + b_ref[...]
    t = logits.shape[0]
    idx = jax.lax.broadcasted_iota(jnp.int32, (t, N_EXPERT), 1)
    big = jnp.int32(N_EXPERT)

    m1 = jnp.max(logits, axis=1, keepdims=True)
    id1 = jnp.min(jnp.where(logits == m1, idx, big), axis=1, keepdims=True)
    masked = jnp.where(idx == id1, _NEG_INF, logits)
    m2 = jnp.max(masked, axis=1, keepdims=True)
    id2 = jnp.min(jnp.where(masked == m2, idx, big), axis=1, keepdims=True)

    s = jnp.exp(m2 - m1)
    denom = 1.0 + s
    p1 = 1.0 / denom
    p2 = s / denom

    out_ref[...] = jnp.where(idx == id1, p1, 0.0) + jnp.where(idx == id2, p2, 0.0)
    ids_ref[...] = jnp.concatenate([id1, id2], axis=1)


def kernel(x, W, b):
    B, S, D = x.shape
    n_tokens = B * S
    xs = x.reshape(n_tokens, D)
    b2 = b.reshape(1, N_EXPERT)
    grid = (n_tokens // TOKEN_BLOCK,)
    out, ids = pl.pallas_call(
        _router_block,
        grid=grid,
        in_specs=[
            pl.BlockSpec((TOKEN_BLOCK, D), lambda i: (i, 0)),
            pl.BlockSpec((N_EXPERT, D), lambda i: (0, 0)),
            pl.BlockSpec((1, N_EXPERT), lambda i: (0, 0)),
        ],
        out_specs=[
            pl.BlockSpec((TOKEN_BLOCK, N_EXPERT), lambda i: (i, 0)),
            pl.BlockSpec((TOKEN_BLOCK, 2), lambda i: (i, 0)),
        ],
        out_shape=[
            jax.ShapeDtypeStruct((n_tokens, N_EXPERT), jnp.float32),
            jax.ShapeDtypeStruct((n_tokens, 2), jnp.int32),
        ],
    )(xs, W, b2)
    return out.reshape(B, S, N_EXPERT), ids.reshape(B, S, 2)


# speedup vs baseline: 7.0400x; 1.0852x over previous
import jax
import jax.numpy as jnp
from jax.experimental import pallas as pl

N_EXPERT = 64
TOKEN_BLOCK = 1024
_NEG_INF = float("-inf")


def _router_block(x_ref, w_ref, b_ref, out_ref, ids_ref):
    nb, t, d = x_ref.shape
    x = x_ref[...].reshape(nb * t, d)
    logits = jax.lax.dot_general(
        x, w_ref[...], (((1,), (1,)), ((), ())),
        preferred_element_type=jnp.float32,
    ) + b_ref[...]
    nt = nb * t
    idx = jax.lax.broadcasted_iota(jnp.int32, (nt, N_EXPERT), 1)
    big = jnp.int32(N_EXPERT)

    m1 = jnp.max(logits, axis=1, keepdims=True)
    id1 = jnp.min(jnp.where(logits == m1, idx, big), axis=1, keepdims=True)
    masked = jnp.where(idx == id1, _NEG_INF, logits)
    m2 = jnp.max(masked, axis=1, keepdims=True)
    id2 = jnp.min(jnp.where(masked == m2, idx, big), axis=1, keepdims=True)

    s = jnp.exp(m2 - m1)
    denom = 1.0 + s
    p1 = 1.0 / denom
    p2 = s / denom

    out = jnp.where(idx == id1, p1, 0.0) + jnp.where(idx == id2, p2, 0.0)
    out_ref[...] = out.reshape(nb, t, N_EXPERT)
    ids_ref[...] = jnp.concatenate([id1, id2], axis=1).reshape(nb, t, 2)


def kernel(x, W, b):
    B, S, D = x.shape
    b2 = b.reshape(1, N_EXPERT)
    grid = (S // TOKEN_BLOCK,)
    out, ids = pl.pallas_call(
        _router_block,
        grid=grid,
        in_specs=[
            pl.BlockSpec((B, TOKEN_BLOCK, D), lambda i: (0, i, 0)),
            pl.BlockSpec((N_EXPERT, D), lambda i: (0, 0)),
            pl.BlockSpec((1, N_EXPERT), lambda i: (0, 0)),
        ],
        out_specs=[
            pl.BlockSpec((B, TOKEN_BLOCK, N_EXPERT), lambda i: (0, i, 0)),
            pl.BlockSpec((B, TOKEN_BLOCK, 2), lambda i: (0, i, 0)),
        ],
        out_shape=[
            jax.ShapeDtypeStruct((B, S, N_EXPERT), jnp.float32),
            jax.ShapeDtypeStruct((B, S, 2), jnp.int32),
        ],
    )(x, W, b2)
    return out, ids
